# baseline (speedup 1.0000x reference)
import functools

import jax
import jax.numpy as jnp
from jax import lax
from jax.experimental import pallas as pl
from jax.experimental.pallas import tpu as pltpu
from jax.experimental.pallas import tpu_sc as plsc

_NC = 2
_NS = 16
_NW = _NC * _NS
_LANES = 16


def _tc_project(tables, w3, vblk):
    nf, vocab, h = tables.shape
    grid_v = vocab // vblk
    q = h // 2

    def body(t_ref, w_ref, o_ref):
        tb = t_ref[0].astype(jnp.bfloat16)
        wb = w_ref[0]
        y = lax.dot_general(
            tb, wb, dimension_numbers=(((1,), (0,)), ((), ())),
            preferred_element_type=jnp.float32)
        u = lax.bitcast_convert_type(y, jnp.uint32)
        r = (u + jnp.uint32(0x7FFF) + ((u >> 16) & jnp.uint32(1))) >> 16
        o_ref[0] = r[:, 0:q] | (r[:, q:2 * q] << 16)

    return pl.pallas_call(
        body,
        grid=(nf, grid_v),
        in_specs=[
            pl.BlockSpec((1, vblk, h), lambda f, j: (f, j, 0)),
            pl.BlockSpec((1, h, h), lambda f, j: (f, 0, 0)),
        ],
        out_specs=pl.BlockSpec((1, vblk, q), lambda f, j: (f, j, 0)),
        out_shape=jax.ShapeDtypeStruct((nf, vocab, q), jnp.uint32),
        compiler_params=pltpu.CompilerParams(
            dimension_semantics=("parallel", "parallel")),
    )(tables, w3)


def _sc_gather_reduce(pflat, idx3, bias4, bsz, h, nf):
    ng, g = idx3.shape[1], idx3.shape[2]
    bpg = g // nf
    b_per_w = bsz // _NW
    hw = h // (2 * _LANES)

    mesh = plsc.VectorSubcoreMesh(core_axis_name="c", subcore_axis_name="s")

    @functools.partial(
        pl.kernel,
        mesh=mesh,
        out_type=jax.ShapeDtypeStruct((bsz * h,), jnp.float32),
        scratch_types=[
            pltpu.VMEM((ng, g), jnp.int32),
            pltpu.VMEM((2, g, h // 2), jnp.float32),
            pltpu.VMEM((b_per_w * h,), jnp.float32),
            pltpu.VMEM((2 * hw, _LANES), jnp.float32),
            pltpu.SemaphoreType.DMA,
        ],
        compiler_params=pltpu.CompilerParams(use_tc_tiling_on_sc=False),
    )
    def k(p_hbm, idx_hbm, bias_hbm, out_hbm, idx_v, buf, ostage, bvec, sem0):
        wid = lax.axis_index("s") * _NC + lax.axis_index("c")
        pltpu.sync_copy(idx_hbm.at[wid], idx_v)
        pltpu.sync_copy(bias_hbm, bvec)

        pltpu.make_async_copy(p_hbm.at[idx_v.at[0]], buf.at[0], sem0).start()

        def step(gi, _):
            par = lax.rem(gi, 2)

            @pl.when(gi + 1 < ng)
            def _():
                pltpu.make_async_copy(
                    p_hbm.at[idx_v.at[gi + 1]], buf.at[1 - par], sem0
                ).start()

            pltpu.make_async_copy(
                p_hbm.at[idx_v.at[gi]], buf.at[par], sem0
            ).wait()

            for bb in range(bpg):
                base = bb * nf
                accs = [None] * (2 * hw)
                for r in range(nf):
                    for c in range(hw):
                        v = buf[par, base + r, pl.ds(c * _LANES, _LANES)]
                        u = lax.bitcast_convert_type(v, jnp.uint32)
                        lo = lax.bitcast_convert_type(u << 16, jnp.float32)
                        hi = lax.bitcast_convert_type(
                            u & jnp.uint32(0xFFFF0000), jnp.float32)
                        for t, part in ((0, lo), (1, hi)):
                            qi = 2 * c + t
                            accs[qi] = part if accs[qi] is None else accs[qi] + part
                rbase = (gi * bpg + bb) * h
                for qi in range(2 * hw):
                    val = jnp.maximum(accs[qi] + bvec[qi], 0.0)
                    ostage[pl.ds(rbase + _LANES * qi, _LANES)] = val
            return 0

        lax.fori_loop(0, ng, step, 0)
        pltpu.sync_copy(
            ostage, out_hbm.at[pl.ds(wid * b_per_w * h, b_per_w * h)])

    return k(pflat, idx3, bias4)


def kernel(x, tables, W, b):
    bsz, nf = x.shape
    _, vocab, h = tables.shape

    hw = h // (2 * _LANES)
    perm = jnp.concatenate([
        jnp.arange(2 * _LANES * c + t * _LANES, 2 * _LANES * c + (t + 1) * _LANES)
        for t in range(2) for c in range(hw)
    ])
    w3 = W.reshape(nf, h, h)[:, :, perm].astype(jnp.bfloat16)

    p3 = _tc_project(tables, w3, 10000)
    pflat = lax.bitcast_convert_type(p3, jnp.float32).reshape(nf * vocab, h // 2)

    f_off = (jnp.arange(nf, dtype=jnp.int32) * vocab)[None, :]
    idx = (x.astype(jnp.int32) + f_off).reshape(-1)
    g = 4 * nf
    ng = (bsz * nf) // (_NW * g)
    idx3 = idx.reshape(_NW, ng, g)

    bias4 = b.reshape(2 * hw, _LANES)

    return _sc_gather_reduce(pflat, idx3, bias4, bsz, h, nf).reshape(bsz, h)


# speedup vs baseline: 1.6663x; 1.6663x over previous
import functools

import jax
import jax.numpy as jnp
from jax import lax
from jax.experimental import pallas as pl
from jax.experimental.pallas import tpu as pltpu
from jax.experimental.pallas import tpu_sc as plsc

_NC = 2
_NS = 16
_NW = _NC * _NS
_LANES = 16


def _tc_project(tt, w3, cblk):
    nf, h, vocab = tt.shape
    nchunk = vocab // cblk
    q = h // 2

    def body(w_ref, t_hbm, o_hbm, tbuf, obuf, isem, osem):
        f = pl.program_id(0)
        wb = w_ref[0]
        fpar = lax.rem(f, 2)

        def copy_in(fi, par):
            return pltpu.make_async_copy(t_hbm.at[fi], tbuf.at[par], isem)

        def copy_out(ci):
            return pltpu.make_async_copy(
                obuf.at[ci % 2], o_hbm.at[pl.ds(f * vocab + ci * cblk, cblk)],
                osem)

        copy_in(f, 0).start()
        copy_in(f, 0).wait()

        @pl.when(f > 0)
        def _():
            copy_out(0).wait()
            copy_out(1).wait()

        for ci in range(nchunk):
            if ci >= 2:
                copy_out(ci - 2).wait()
            tb = tbuf[0, :, pl.ds(ci * cblk, cblk)].astype(jnp.bfloat16)
            y = lax.dot_general(
                tb, wb, dimension_numbers=(((0,), (0,)), ((), ())),
                preferred_element_type=jnp.float32)
            u = lax.bitcast_convert_type(y, jnp.uint32)
            p = (u[:, 0:q] >> 16) | (u[:, q:2 * q] & jnp.uint32(0xFFFF0000))
            obuf[ci % 2] = lax.bitcast_convert_type(p, jnp.float32)
            copy_out(ci).start()

        @pl.when(f == nf - 1)
        def _():
            copy_out(nchunk - 2).wait()
            copy_out(nchunk - 1).wait()

    return pl.pallas_call(
        body,
        grid=(nf,),
        in_specs=[
            pl.BlockSpec((1, h, h), lambda f: (f, 0, 0)),
            pl.BlockSpec(memory_space=pltpu.MemorySpace.HBM),
        ],
        out_specs=pl.BlockSpec(memory_space=pltpu.MemorySpace.HBM),
        out_shape=jax.ShapeDtypeStruct((nf * vocab, q), jnp.float32),
        scratch_shapes=[
            pltpu.VMEM((1, h, vocab), jnp.float32),
            pltpu.VMEM((2, cblk, q), jnp.float32),
            pltpu.SemaphoreType.DMA,
            pltpu.SemaphoreType.DMA,
        ],
        compiler_params=pltpu.CompilerParams(
            dimension_semantics=("arbitrary",)),
    )(w3, tt)


def _sc_gather_reduce(pflat, idx3, bias4, bsz, h, nf):
    ng, g = idx3.shape[1], idx3.shape[2]
    bpg = g // nf
    b_per_w = bsz // _NW
    hw = h // (2 * _LANES)

    mesh = plsc.VectorSubcoreMesh(core_axis_name="c", subcore_axis_name="s")

    @functools.partial(
        pl.kernel,
        mesh=mesh,
        out_type=jax.ShapeDtypeStruct((bsz * h,), jnp.float32),
        scratch_types=[
            pltpu.VMEM((ng, g), jnp.int32),
            pltpu.VMEM((2, g, h // 2), jnp.float32),
            pltpu.VMEM((b_per_w * h,), jnp.float32),
            pltpu.VMEM((2 * hw, _LANES), jnp.float32),
            pltpu.SemaphoreType.DMA,
        ],
        compiler_params=pltpu.CompilerParams(use_tc_tiling_on_sc=False),
    )
    def k(p_hbm, idx_hbm, bias_hbm, out_hbm, idx_v, buf, ostage, bvec, sem0):
        wid = lax.axis_index("s") * _NC + lax.axis_index("c")
        pltpu.sync_copy(idx_hbm.at[wid], idx_v)
        pltpu.sync_copy(bias_hbm, bvec)

        pltpu.make_async_copy(p_hbm.at[idx_v.at[0]], buf.at[0], sem0).start()

        def step(gi, _):
            par = lax.rem(gi, 2)

            @pl.when(gi + 1 < ng)
            def _():
                pltpu.make_async_copy(
                    p_hbm.at[idx_v.at[gi + 1]], buf.at[1 - par], sem0
                ).start()

            pltpu.make_async_copy(
                p_hbm.at[idx_v.at[gi]], buf.at[par], sem0
            ).wait()

            for bb in range(bpg):
                base = bb * nf
                accs = [None] * (2 * hw)
                for r in range(nf):
                    for c in range(hw):
                        v = buf[par, base + r, pl.ds(c * _LANES, _LANES)]
                        u = lax.bitcast_convert_type(v, jnp.uint32)
                        lo = lax.bitcast_convert_type(u << 16, jnp.float32)
                        hi = lax.bitcast_convert_type(
                            u & jnp.uint32(0xFFFF0000), jnp.float32)
                        for t, part in ((0, lo), (1, hi)):
                            qi = 2 * c + t
                            accs[qi] = part if accs[qi] is None else accs[qi] + part
                rbase = (gi * bpg + bb) * h
                for qi in range(2 * hw):
                    val = jnp.maximum(accs[qi] + bvec[qi], 0.0)
                    ostage[pl.ds(rbase + _LANES * qi, _LANES)] = val
            return 0

        lax.fori_loop(0, ng, step, 0)
        pltpu.sync_copy(
            ostage, out_hbm.at[pl.ds(wid * b_per_w * h, b_per_w * h)])

    return k(pflat, idx3, bias4)


def kernel(x, tables, W, b):
    bsz, nf = x.shape
    _, vocab, h = tables.shape

    tt = jnp.transpose(tables, (0, 2, 1))

    hw = h // (2 * _LANES)
    perm = jnp.concatenate([
        jnp.arange(2 * _LANES * c + t * _LANES, 2 * _LANES * c + (t + 1) * _LANES)
        for t in range(2) for c in range(hw)
    ])
    w3 = W.reshape(nf, h, h)[:, :, perm].astype(jnp.bfloat16)

    pflat = _tc_project(tt, w3, 5000)

    f_off = (jnp.arange(nf, dtype=jnp.int32) * vocab)[None, :]
    idx = (x.astype(jnp.int32) + f_off).reshape(-1)
    g = 4 * nf
    ng = (bsz * nf) // (_NW * g)
    idx3 = idx.reshape(_NW, ng, g)

    bias4 = b.reshape(2 * hw, _LANES)

    return _sc_gather_reduce(pflat, idx3, bias4, bsz, h, nf).reshape(bsz, h)


# speedup vs baseline: 2.7256x; 1.6357x over previous
import functools

import jax
import jax.numpy as jnp
from jax import lax
from jax.experimental import pallas as pl
from jax.experimental.pallas import tpu as pltpu
from jax.experimental.pallas import tpu_sc as plsc

_NC = 2
_NS = 16
_NW = _NC * _NS
_LANES = 16


def _tc_project_pairs(tt2, w2, vocab, vblk):
    np_, kk, _ = tt2.shape
    grid_v = (vocab + vblk - 1) // vblk
    q = kk // 4

    def body(tt_ref, w_ref, o_ref):
        ttb = tt_ref[0].astype(jnp.bfloat16)
        wb = w_ref[0]
        y = lax.dot_general(
            ttb, wb, dimension_numbers=(((0,), (0,)), ((), ())),
            preferred_element_type=jnp.float32)
        u = lax.bitcast_convert_type(y, jnp.uint32)
        r = (u + jnp.uint32(0x7FFF) + ((u >> 16) & jnp.uint32(1))) >> 16
        o_ref[0, :, 0:q] = lax.bitcast_convert_type(
            r[:, 0:q] | (r[:, q:2 * q] << 16), jnp.float32)
        o_ref[0, :, q:2 * q] = lax.bitcast_convert_type(
            r[:, 2 * q:3 * q] | (r[:, 3 * q:4 * q] << 16), jnp.float32)

    return pl.pallas_call(
        body,
        grid=(np_, grid_v),
        in_specs=[
            pl.BlockSpec((1, kk, vblk), lambda p, j: (p, 0, j)),
            pl.BlockSpec((1, kk, kk), lambda p, j: (p, 0, 0)),
        ],
        out_specs=pl.BlockSpec((1, vblk, kk // 2), lambda p, j: (p, j, 0)),
        out_shape=jax.ShapeDtypeStruct((np_, vocab, kk // 2), jnp.float32),
        compiler_params=pltpu.CompilerParams(
            dimension_semantics=("parallel", "parallel")),
    )(tt2, w2)


def _sc_gather_reduce(pflat, idx3, bias4, bsz, h, nf):
    ng, g = idx3.shape[1], idx3.shape[2]
    bpg = g // nf
    b_per_w = bsz // _NW
    hw = h // (2 * _LANES)

    mesh = plsc.VectorSubcoreMesh(core_axis_name="c", subcore_axis_name="s")

    @functools.partial(
        pl.kernel,
        mesh=mesh,
        out_type=jax.ShapeDtypeStruct((bsz * h,), jnp.float32),
        scratch_types=[
            pltpu.VMEM((ng, g), jnp.int32),
            pltpu.VMEM((2, g, h // 2), jnp.float32),
            pltpu.VMEM((b_per_w * h,), jnp.float32),
            pltpu.VMEM((2 * hw, _LANES), jnp.float32),
            pltpu.SemaphoreType.DMA,
        ],
        compiler_params=pltpu.CompilerParams(use_tc_tiling_on_sc=False),
    )
    def k(p_hbm, idx_hbm, bias_hbm, out_hbm, idx_v, buf, ostage, bvec, sem0):
        wid = lax.axis_index("s") * _NC + lax.axis_index("c")
        pltpu.sync_copy(idx_hbm.at[wid], idx_v)
        pltpu.sync_copy(bias_hbm, bvec)

        pltpu.make_async_copy(p_hbm.at[idx_v.at[0]], buf.at[0], sem0).start()

        def step(gi, _):
            par = lax.rem(gi, 2)

            @pl.when(gi + 1 < ng)
            def _():
                pltpu.make_async_copy(
                    p_hbm.at[idx_v.at[gi + 1]], buf.at[1 - par], sem0
                ).start()

            pltpu.make_async_copy(
                p_hbm.at[idx_v.at[gi]], buf.at[par], sem0
            ).wait()

            for bb in range(bpg):
                base = bb * nf
                accs = [None] * (2 * hw)
                for r in range(nf):
                    for c in range(hw):
                        v = buf[par, base + r, pl.ds(c * _LANES, _LANES)]
                        u = lax.bitcast_convert_type(v, jnp.uint32)
                        lo = lax.bitcast_convert_type(u << 16, jnp.float32)
                        hi = lax.bitcast_convert_type(
                            u & jnp.uint32(0xFFFF0000), jnp.float32)
                        for t, part in ((0, lo), (1, hi)):
                            qi = 2 * c + t
                            accs[qi] = part if accs[qi] is None else accs[qi] + part
                rbase = (gi * bpg + bb) * h
                for qi in range(2 * hw):
                    val = jnp.maximum(accs[qi] + bvec[qi], 0.0)
                    ostage[pl.ds(rbase + _LANES * qi, _LANES)] = val
            return 0

        lax.fori_loop(0, ng, step, 0)
        pltpu.sync_copy(
            ostage, out_hbm.at[pl.ds(wid * b_per_w * h, b_per_w * h)])

    return k(pflat, idx3, bias4)


def kernel(x, tables, W, b):
    bsz, nf = x.shape
    _, vocab, h = tables.shape
    npair = nf // 2

    tt2 = jnp.transpose(tables, (0, 2, 1)).reshape(npair, 2 * h, vocab)

    hw = h // (2 * _LANES)
    perm = jnp.concatenate([
        jnp.arange(2 * _LANES * c + t * _LANES, 2 * _LANES * c + (t + 1) * _LANES)
        for t in range(2) for c in range(hw)
    ])
    W3 = W.reshape(nf, h, h)[:, :, perm]
    w2 = jnp.zeros((npair, 2 * h, 2 * h), jnp.float32)
    w2 = w2.at[:, 0:h, 0:h].set(W3[0::2])
    w2 = w2.at[:, h:2 * h, h:2 * h].set(W3[1::2])
    w2 = w2.astype(jnp.bfloat16)

    p2 = _tc_project_pairs(tt2, w2, vocab, 4096)
    pflat = p2.reshape(nf * vocab, h // 2)

    f = jnp.arange(nf, dtype=jnp.int32)
    offs = ((f // 2) * (2 * vocab) + (f & 1))[None, :]
    idx = (2 * x.astype(jnp.int32) + offs).reshape(-1)
    g = 4 * nf
    ng = (bsz * nf) // (_NW * g)
    idx3 = idx.reshape(_NW, ng, g)

    bias4 = b.reshape(2 * hw, _LANES)

    return _sc_gather_reduce(pflat, idx3, bias4, bsz, h, nf).reshape(bsz, h)
